# baseline (speedup 1.0000x reference)
import functools

import jax
import jax.numpy as jnp
from jax import lax
from jax.experimental import pallas as pl
from jax.experimental.pallas import tpu as pltpu
from jax.experimental.pallas import tpu_sc as plsc

NC = 2
NS = 16
EB1 = 800
EB2 = 2000


def _round_up(a, b):
    return (a + b - 1) // b * b


def _sc_mesh():
    return plsc.VectorSubcoreMesh(
        core_axis_name="c", subcore_axis_name="s", num_cores=NC, num_subcores=NS
    )


_SC_PARAMS = pltpu.CompilerParams(use_tc_tiling_on_sc=False)


def _make_deg_kernel(NP, EP, D, EB):
    EW = EP // (NC * NS)
    NB = EW // EB
    RB = NP // NS

    @functools.partial(
        pl.kernel,
        out_type=jax.ShapeDtypeStruct((NC * NP, D), jnp.float32),
        mesh=_sc_mesh(),
        scratch_types=[
            pltpu.VMEM((EB,), jnp.int32),
            pltpu.VMEM((EB,), jnp.int32),
            pltpu.VMEM((EB, D), jnp.float32),
            pltpu.VMEM_SHARED((NP, D), jnp.float32),
            pltpu.SemaphoreType.DMA,
            pltpu.SemaphoreType.DMA,
        ],
        compiler_params=_SC_PARAMS,
    )
    def deg_kernel(dst1_hbm, ones_hbm, z_hbm, out_hbm, idst0, idst1, ones_v,
                   acc, lsem, ssem):
        c = lax.axis_index("c")
        s = lax.axis_index("s")
        wid = s * NC + c
        pltpu.sync_copy(z_hbm.at[pl.ds(s * RB, RB)], acc.at[pl.ds(s * RB, RB)])
        pltpu.sync_copy(ones_hbm, ones_v)
        plsc.subcore_barrier()

        base = wid * EW
        pltpu.sync_copy(dst1_hbm.at[pl.ds(base, EB)], idst0)

        @pl.loop(0, NB // 2)
        def _(t):
            for p in (0, 1):
                b = 2 * t + p
                idsp, idsq = (idst0, idst1) if p == 0 else (idst1, idst0)

                @pl.when(b > 0)
                def _():
                    pltpu.make_async_copy(ones_v, acc.at[idsq], ssem).wait()

                @pl.when(b + 1 < NB)
                def _():
                    pltpu.async_copy(
                        dst1_hbm.at[pl.ds(base + (b + 1) * EB, EB)],
                        idsq, lsem,
                    )

                pltpu.async_copy(ones_v, acc.at[idsp], ssem, add=True)

                @pl.when(b + 1 < NB)
                def _():
                    pltpu.make_async_copy(
                        dst1_hbm.at[pl.ds(0, EB)], idsq, lsem
                    ).wait()

        pltpu.make_async_copy(ones_v, acc.at[idst1], ssem).wait()

        plsc.subcore_barrier()
        pltpu.sync_copy(
            acc.at[pl.ds(s * RB, RB)], out_hbm.at[pl.ds(c * NP + s * RB, RB)]
        )

    return deg_kernel


def _make_agg_kernel(NP, EP, D, EB):
    EW = EP // (NC * NS)
    NB = EW // EB
    RB = NP // NS

    @functools.partial(
        pl.kernel,
        out_type=jax.ShapeDtypeStruct((NC * NP, D), jnp.float32),
        mesh=_sc_mesh(),
        scratch_types=[
            pltpu.VMEM((EB,), jnp.int32),
            pltpu.VMEM((EB,), jnp.int32),
            pltpu.VMEM((EB,), jnp.int32),
            pltpu.VMEM((EB,), jnp.int32),
            pltpu.VMEM((EB, D), jnp.float32),
            pltpu.VMEM((EB, D), jnp.float32),
            pltpu.VMEM_SHARED((NP, D), jnp.float32),
            pltpu.SemaphoreType.DMA,
            pltpu.SemaphoreType.DMA,
            pltpu.SemaphoreType.DMA,
        ],
        compiler_params=_SC_PARAMS,
    )
    def agg_kernel(src1_hbm, dst1_hbm, u_hbm, z_hbm, out_hbm,
                   isrc0, isrc1, idst0, idst1, rows0, rows1, acc,
                   lsem, gsem, ssem):
        c = lax.axis_index("c")
        s = lax.axis_index("s")
        wid = s * NC + c
        pltpu.sync_copy(z_hbm.at[pl.ds(s * RB, RB)], acc.at[pl.ds(s * RB, RB)])
        plsc.subcore_barrier()

        base = wid * EW
        pltpu.sync_copy(src1_hbm.at[pl.ds(base, EB)], isrc0)
        pltpu.sync_copy(dst1_hbm.at[pl.ds(base, EB)], idst0)
        pltpu.async_copy(u_hbm.at[isrc0], rows0, gsem)

        @pl.loop(0, NB // 2)
        def _(t):
            for p in (0, 1):
                b = 2 * t + p
                isp, isq = (isrc0, isrc1) if p == 0 else (isrc1, isrc0)
                idsp, idsq = (idst0, idst1) if p == 0 else (idst1, idst0)
                rsp, rsq = (rows0, rows1) if p == 0 else (rows1, rows0)

                @pl.when(b > 0)
                def _():
                    pltpu.make_async_copy(rsq, acc.at[idsq], ssem).wait()

                @pl.when(b + 1 < NB)
                def _():
                    pltpu.async_copy(
                        src1_hbm.at[pl.ds(base + (b + 1) * EB, EB)],
                        isq, lsem,
                    )
                    pltpu.async_copy(
                        dst1_hbm.at[pl.ds(base + (b + 1) * EB, EB)],
                        idsq, lsem,
                    )

                pltpu.make_async_copy(u_hbm.at[isp], rsp, gsem).wait()
                pltpu.async_copy(rsp, acc.at[idsp], ssem, add=True)

                @pl.when(b + 1 < NB)
                def _():
                    pltpu.make_async_copy(
                        src1_hbm.at[pl.ds(0, EB)], isq, lsem
                    ).wait()
                    pltpu.make_async_copy(
                        dst1_hbm.at[pl.ds(0, EB)], idsq, lsem
                    ).wait()
                    pltpu.async_copy(u_hbm.at[isq], rsq, gsem)

        if NB % 2:
            pltpu.make_async_copy(rows1, acc.at[idst1], ssem).wait()
            pltpu.make_async_copy(u_hbm.at[isrc0], rows0, gsem).wait()
            pltpu.async_copy(rows0, acc.at[idst0], ssem, add=True)
            pltpu.make_async_copy(rows0, acc.at[idst0], ssem).wait()
        else:
            pltpu.make_async_copy(rows1, acc.at[idst1], ssem).wait()

        plsc.subcore_barrier()
        pltpu.sync_copy(
            acc.at[pl.ds(s * RB, RB)], out_hbm.at[pl.ds(c * NP + s * RB, RB)]
        )

    return agg_kernel


def _tc1_body(d0, d1, x, e16, w, o):
    s = d0[...] + d1[...]
    dv16 = lax.rsqrt(
        jnp.dot(s, e16[...], preferred_element_type=jnp.float32) + 1.0)
    o[...] = jnp.dot(x[...], w[...],
                     preferred_element_type=jnp.float32) * dv16


def _tc2_body(d0, d1, a0, a1, u1, e16, w, b, o):
    s = d0[...] + d1[...]
    dv16 = lax.rsqrt(
        jnp.dot(s, e16[...], preferred_element_type=jnp.float32) + 1.0)
    dv8 = lax.rsqrt(s + 1.0)
    t = dv16 * (a0[...] + a1[...] + u1[...]) + b[...]
    r = jnp.maximum(t, 0.0)
    o[...] = jnp.dot(r, w[...], preferred_element_type=jnp.float32) * dv8


def _tc3_body(d0, d1, c0, c1, u2, b, o):
    dv8 = lax.rsqrt(d0[...] + d1[...] + 1.0)
    o[...] = dv8 * (c0[...] + c1[...] + u2[...]) + b[...]


def _half(bh, w, off):
    return pl.BlockSpec((bh, w), lambda i, o=off: (i + o, 0))


def _mat(bh, w):
    return pl.BlockSpec((bh, w), lambda i: (i, 0))


def _full(shape):
    return pl.BlockSpec(shape, lambda i: tuple(0 for _ in shape))


def kernel(x, edge_index, W1, b1, W2, b2):
    N, IN_D = x.shape
    HID = W1.shape[1]
    OUT = W2.shape[1]
    E = edge_index.shape[1]

    NP = _round_up(N + 1, NS * 8 * 56)
    QR = NP // 16
    GRIDQ = 7
    BH = QR // GRIDQ
    EP = _round_up(E, NC * NS * 2 * EB1 * EB2 // 400)

    f32 = jnp.float32
    ei = edge_index.astype(jnp.int32)
    if EP > E:
        ei = jnp.concatenate([ei, jnp.full((2, EP - E), N, jnp.int32)], axis=1)
    src1 = ei[0]
    dst1 = ei[1]
    z16 = jnp.zeros((NP, HID), f32)
    z8 = jnp.zeros((NP, OUT), f32)
    ones8 = jnp.ones((EB2, OUT), f32)
    x256 = x.astype(f32).reshape(N * IN_D // 256, 256)
    eye16 = jnp.eye(16, dtype=f32)
    E16 = jnp.kron(eye16, jnp.ones((OUT, HID), f32) / OUT)
    W1bd = jnp.kron(eye16, W1.astype(f32))
    W2bd = jnp.kron(eye16, W2.astype(f32))
    b1t = jnp.tile(b1.astype(f32), (16,)).reshape(1, 16 * HID)
    b2t = jnp.tile(b2.astype(f32), (16,)).reshape(1, 16 * OUT)

    degb = _make_deg_kernel(NP, EP, OUT, EB2)(dst1, ones8, z8)
    degq = degb.reshape(NC * QR, 16 * OUT)

    u1 = pl.pallas_call(
        _tc1_body,
        grid=(GRIDQ,),
        in_specs=[_half(BH, 128, 0), _half(BH, 128, GRIDQ), _mat(BH, 256),
                  _full((128, 256)), _full((256, 256))],
        out_specs=_mat(BH, 256),
        out_shape=jax.ShapeDtypeStruct((QR, 256), f32),
    )(degq, degq, x256, E16, W1bd)

    agg1 = _make_agg_kernel(NP, EP, HID, EB1)(src1, dst1, u1.reshape(NP, HID),
                                            z16)
    agg1q = agg1.reshape(NC * QR, 16 * HID)

    u2 = pl.pallas_call(
        _tc2_body,
        grid=(GRIDQ,),
        in_specs=[_half(BH, 128, 0), _half(BH, 128, GRIDQ),
                  _half(BH, 256, 0), _half(BH, 256, GRIDQ), _mat(BH, 256),
                  _full((128, 256)), _full((256, 128)), _full((1, 256))],
        out_specs=_mat(BH, 128),
        out_shape=jax.ShapeDtypeStruct((QR, 128), f32),
    )(degq, degq, agg1q, agg1q, u1, E16, W2bd, b1t)

    agg2 = _make_agg_kernel(NP, EP, OUT, EB2)(src1, dst1, u2.reshape(NP, OUT),
                                              z8)
    agg2q = agg2.reshape(NC * QR, 16 * OUT)

    opk = pl.pallas_call(
        _tc3_body,
        grid=(GRIDQ,),
        in_specs=[_half(BH, 128, 0), _half(BH, 128, GRIDQ),
                  _half(BH, 128, 0), _half(BH, 128, GRIDQ), _mat(BH, 128),
                  _full((1, 128))],
        out_specs=_mat(BH, 128),
        out_shape=jax.ShapeDtypeStruct((QR, 128), f32),
    )(degq, degq, agg2q, agg2q, u2, b2t)

    return opk[:N * OUT // 128].reshape(N, OUT)


# speedup vs baseline: 2.7735x; 2.7735x over previous
import functools

import jax
import jax.numpy as jnp
from jax import lax
from jax.experimental import pallas as pl
from jax.experimental.pallas import tpu as pltpu
from jax.experimental.pallas import tpu_sc as plsc

NC = 2
NS = 16
EB1 = 400
EB2 = 2000


def _round_up(a, b):
    return (a + b - 1) // b * b


def _sc_mesh():
    return plsc.VectorSubcoreMesh(
        core_axis_name="c", subcore_axis_name="s", num_cores=NC, num_subcores=NS
    )


_SC_PARAMS = pltpu.CompilerParams(use_tc_tiling_on_sc=False)


def _make_deg_kernel(NP, EP, D, EB):
    EW = EP // (NC * NS)
    NB = EW // EB
    RB = NP // NS

    @functools.partial(
        pl.kernel,
        out_type=jax.ShapeDtypeStruct((NC * NP, D), jnp.float32),
        mesh=_sc_mesh(),
        scratch_types=[
            pltpu.VMEM((EB,), jnp.int32),
            pltpu.VMEM((EB,), jnp.int32),
            pltpu.VMEM((EB, D), jnp.float32),
            pltpu.VMEM_SHARED((NP, D), jnp.float32),
            pltpu.SemaphoreType.DMA,
            pltpu.SemaphoreType.DMA,
        ],
        compiler_params=_SC_PARAMS,
    )
    def deg_kernel(dst1_hbm, ones_hbm, z_hbm, out_hbm, idst0, idst1, ones_v,
                   acc, lsem, ssem):
        c = lax.axis_index("c")
        s = lax.axis_index("s")
        wid = s * NC + c
        pltpu.sync_copy(z_hbm.at[pl.ds(s * RB, RB)], acc.at[pl.ds(s * RB, RB)])
        pltpu.sync_copy(ones_hbm, ones_v)
        plsc.subcore_barrier()

        base = wid * EW
        pltpu.sync_copy(dst1_hbm.at[pl.ds(base, EB)], idst0)

        @pl.loop(0, NB // 2)
        def _(t):
            for p in (0, 1):
                b = 2 * t + p
                idsp, idsq = (idst0, idst1) if p == 0 else (idst1, idst0)

                @pl.when(b > 0)
                def _():
                    pltpu.make_async_copy(ones_v, acc.at[idsq], ssem).wait()

                @pl.when(b + 1 < NB)
                def _():
                    pltpu.async_copy(
                        dst1_hbm.at[pl.ds(base + (b + 1) * EB, EB)],
                        idsq, lsem,
                    )

                pltpu.async_copy(ones_v, acc.at[idsp], ssem, add=True)

                @pl.when(b + 1 < NB)
                def _():
                    pltpu.make_async_copy(
                        dst1_hbm.at[pl.ds(0, EB)], idsq, lsem
                    ).wait()

        pltpu.make_async_copy(ones_v, acc.at[idst1], ssem).wait()

        plsc.subcore_barrier()
        pltpu.sync_copy(
            acc.at[pl.ds(s * RB, RB)], out_hbm.at[pl.ds(c * NP + s * RB, RB)]
        )

    return deg_kernel


def _make_agg_kernel(NP, EP, D, EB):
    EW = EP // (NC * NS)
    NB = EW // EB
    RB = NP // NS

    @functools.partial(
        pl.kernel,
        out_type=jax.ShapeDtypeStruct((NC * NP, D), jnp.float32),
        mesh=_sc_mesh(),
        scratch_types=[
            pltpu.VMEM((EB,), jnp.int32),
            pltpu.VMEM((EB,), jnp.int32),
            pltpu.VMEM((EB,), jnp.int32),
            pltpu.VMEM((EB,), jnp.int32),
            pltpu.VMEM((EB, D), jnp.float32),
            pltpu.VMEM((EB, D), jnp.float32),
            pltpu.VMEM_SHARED((NP, D), jnp.float32),
            pltpu.SemaphoreType.DMA,
            pltpu.SemaphoreType.DMA,
            pltpu.SemaphoreType.DMA,
        ],
        compiler_params=_SC_PARAMS,
    )
    def agg_kernel(src1_hbm, dst1_hbm, u_hbm, z_hbm, out_hbm,
                   isrc0, isrc1, idst0, idst1, rows0, rows1, acc,
                   lsem, gsem, ssem):
        c = lax.axis_index("c")
        s = lax.axis_index("s")
        wid = s * NC + c
        pltpu.sync_copy(z_hbm.at[pl.ds(s * RB, RB)], acc.at[pl.ds(s * RB, RB)])
        plsc.subcore_barrier()

        base = wid * EW
        pltpu.sync_copy(src1_hbm.at[pl.ds(base, EB)], isrc0)
        pltpu.sync_copy(dst1_hbm.at[pl.ds(base, EB)], idst0)
        pltpu.async_copy(u_hbm.at[isrc0], rows0, gsem)

        @pl.loop(0, NB // 2)
        def _(t):
            for p in (0, 1):
                b = 2 * t + p
                isp, isq = (isrc0, isrc1) if p == 0 else (isrc1, isrc0)
                idsp, idsq = (idst0, idst1) if p == 0 else (idst1, idst0)
                rsp, rsq = (rows0, rows1) if p == 0 else (rows1, rows0)

                @pl.when(b > 0)
                def _():
                    pltpu.make_async_copy(rsq, acc.at[idsq], ssem).wait()

                @pl.when(b + 1 < NB)
                def _():
                    pltpu.async_copy(
                        src1_hbm.at[pl.ds(base + (b + 1) * EB, EB)],
                        isq, lsem,
                    )
                    pltpu.async_copy(
                        dst1_hbm.at[pl.ds(base + (b + 1) * EB, EB)],
                        idsq, lsem,
                    )

                pltpu.make_async_copy(u_hbm.at[isp], rsp, gsem).wait()
                pltpu.async_copy(rsp, acc.at[idsp], ssem, add=True)

                @pl.when(b + 1 < NB)
                def _():
                    pltpu.make_async_copy(
                        src1_hbm.at[pl.ds(0, EB)], isq, lsem
                    ).wait()
                    pltpu.make_async_copy(
                        dst1_hbm.at[pl.ds(0, EB)], idsq, lsem
                    ).wait()
                    pltpu.async_copy(u_hbm.at[isq], rsq, gsem)

        if NB % 2:
            pltpu.make_async_copy(rows1, acc.at[idst1], ssem).wait()
            pltpu.make_async_copy(u_hbm.at[isrc0], rows0, gsem).wait()
            pltpu.async_copy(rows0, acc.at[idst0], ssem, add=True)
            pltpu.make_async_copy(rows0, acc.at[idst0], ssem).wait()
        else:
            pltpu.make_async_copy(rows1, acc.at[idst1], ssem).wait()

        plsc.subcore_barrier()
        pltpu.sync_copy(
            acc.at[pl.ds(s * RB, RB)], out_hbm.at[pl.ds(c * NP + s * RB, RB)]
        )

    return agg_kernel


def _tc1_body(d0, d1, x, e16, w, o):
    s = d0[...] + d1[...]
    dv16 = lax.rsqrt(
        jnp.dot(s, e16[...], preferred_element_type=jnp.float32) + 1.0)
    o[...] = jnp.dot(x[...], w[...],
                     preferred_element_type=jnp.float32) * dv16


def _tc2_body(d0, d1, a0, a1, u1, e16, w, b, o):
    s = d0[...] + d1[...]
    dv16 = lax.rsqrt(
        jnp.dot(s, e16[...], preferred_element_type=jnp.float32) + 1.0)
    dv8 = lax.rsqrt(s + 1.0)
    t = dv16 * (a0[...] + a1[...] + u1[...]) + b[...]
    r = jnp.maximum(t, 0.0)
    o[...] = jnp.dot(r, w[...], preferred_element_type=jnp.float32) * dv8


def _tc3_body(d0, d1, c0, c1, u2, b, o):
    dv8 = lax.rsqrt(d0[...] + d1[...] + 1.0)
    o[...] = dv8 * (c0[...] + c1[...] + u2[...]) + b[...]


def _half(bh, w, off):
    return pl.BlockSpec((bh, w), lambda i, o=off: (i + o, 0))


def _mat(bh, w):
    return pl.BlockSpec((bh, w), lambda i: (i, 0))


def _full(shape):
    return pl.BlockSpec(shape, lambda i: tuple(0 for _ in shape))


def kernel(x, edge_index, W1, b1, W2, b2):
    N, IN_D = x.shape
    HID = W1.shape[1]
    OUT = W2.shape[1]
    E = edge_index.shape[1]

    NP = _round_up(N + 1, NS * 8 * 56)
    QR = NP // 16
    GRIDQ = 7
    BH = QR // GRIDQ
    EP = _round_up(E, NC * NS * 2 * EB1 * EB2 // 400)

    f32 = jnp.float32
    ei = edge_index.astype(jnp.int32)
    if EP > E:
        ei = jnp.concatenate([ei, jnp.full((2, EP - E), N, jnp.int32)], axis=1)
    src1 = ei[0]
    dst1 = ei[1]
    z16 = jnp.zeros((NP, HID), f32)
    z8 = jnp.zeros((NP, OUT), f32)
    ones8 = jnp.ones((EB2, OUT), f32)
    x256 = x.astype(f32).reshape(N * IN_D // 256, 256)
    eye16 = jnp.eye(16, dtype=f32)
    E16 = jnp.kron(eye16, jnp.ones((OUT, HID), f32) / OUT)
    W1bd = jnp.kron(eye16, W1.astype(f32))
    W2bd = jnp.kron(eye16, W2.astype(f32))
    b1t = jnp.tile(b1.astype(f32), (16,)).reshape(1, 16 * HID)
    b2t = jnp.tile(b2.astype(f32), (16,)).reshape(1, 16 * OUT)

    degb = _make_deg_kernel(NP, EP, OUT, EB2)(dst1, ones8, z8)
    degq = degb.reshape(NC * QR, 16 * OUT)

    u1 = pl.pallas_call(
        _tc1_body,
        grid=(GRIDQ,),
        in_specs=[_half(BH, 128, 0), _half(BH, 128, GRIDQ), _mat(BH, 256),
                  _full((128, 256)), _full((256, 256))],
        out_specs=_mat(BH, 256),
        out_shape=jax.ShapeDtypeStruct((QR, 256), f32),
    )(degq, degq, x256, E16, W1bd)

    agg1 = _make_agg_kernel(NP, EP, HID, EB1)(src1, dst1, u1.reshape(NP, HID),
                                            z16)
    agg1q = agg1.reshape(NC * QR, 16 * HID)

    u2 = pl.pallas_call(
        _tc2_body,
        grid=(GRIDQ,),
        in_specs=[_half(BH, 128, 0), _half(BH, 128, GRIDQ),
                  _half(BH, 256, 0), _half(BH, 256, GRIDQ), _mat(BH, 256),
                  _full((128, 256)), _full((256, 128)), _full((1, 256))],
        out_specs=_mat(BH, 128),
        out_shape=jax.ShapeDtypeStruct((QR, 128), f32),
    )(degq, degq, agg1q, agg1q, u1, E16, W2bd, b1t)

    agg2 = _make_agg_kernel(NP, EP, OUT, EB2)(src1, dst1, u2.reshape(NP, OUT),
                                              z8)
    agg2q = agg2.reshape(NC * QR, 16 * OUT)

    opk = pl.pallas_call(
        _tc3_body,
        grid=(GRIDQ,),
        in_specs=[_half(BH, 128, 0), _half(BH, 128, GRIDQ),
                  _half(BH, 128, 0), _half(BH, 128, GRIDQ), _mat(BH, 128),
                  _full((1, 128))],
        out_specs=_mat(BH, 128),
        out_shape=jax.ShapeDtypeStruct((QR, 128), f32),
    )(degq, degq, agg2q, agg2q, u2, b2t)

    return opk[:N * OUT // 128].reshape(N, OUT)
